# speedup vs baseline: 182.5253x; 182.5253x over previous
import functools

import jax
import jax.numpy as jnp
from jax import lax
from jax.experimental import pallas as pl
from jax.experimental.pallas import tpu as pltpu
from jax.experimental.pallas import tpu_sc as plsc

N_NODES = 100000
N_EDGES = 3200000
NPAD = 100352
ROWS = NPAD // 128
NC, NS = 2, 16
NW = NC * NS
PER_W = N_EDGES // NW
CH = 2000
NCH = PER_W // CH
SEG = NPAD // NS

_MESH = dict(core_axis_name="c", subcore_axis_name="s", num_cores=NC,
             num_subcores=NS)


@functools.partial(
    pl.kernel,
    out_type=jax.ShapeDtypeStruct((NC, NPAD), jnp.float32),
    mesh=plsc.VectorSubcoreMesh(**_MESH),
    scratch_types=[
        pltpu.VMEM((CH,), jnp.int32),
        pltpu.VMEM((CH,), jnp.float32),
        pltpu.VMEM_SHARED((NPAD,), jnp.float32),
    ],
)
def _deg_kernel(dst_hbm, zeros_hbm, ones_hbm, out_hbm, idx_v, ones_v, acc_s):
    cid = lax.axis_index("c")
    sid = lax.axis_index("s")
    wid = sid * NC + cid
    sl = pl.ds(sid * SEG, SEG)
    pltpu.sync_copy(zeros_hbm.at[sl], acc_s.at[sl])
    pltpu.sync_copy(ones_hbm, ones_v)
    plsc.subcore_barrier()
    base = wid * PER_W

    def body(i, carry):
        off = base + i * CH
        pltpu.sync_copy(dst_hbm.at[pl.ds(off, CH)], idx_v)
        pltpu.sync_copy(ones_v, acc_s.at[idx_v], add=True)
        return carry

    lax.fori_loop(0, NCH, body, 0)
    plsc.subcore_barrier()
    pltpu.sync_copy(acc_s.at[sl], out_hbm.at[cid, sl])


def _make_prop(F):
    scratch = [pltpu.VMEM((CH,), jnp.int32), pltpu.VMEM((CH,), jnp.int32)]
    scratch += [pltpu.VMEM((CH,), jnp.float32) for _ in range(F)]
    scratch += [pltpu.VMEM_SHARED((NPAD,), jnp.float32) for _ in range(2 * F)]

    @functools.partial(
        pl.kernel,
        out_type=jax.ShapeDtypeStruct((NC, F, NPAD), jnp.float32),
        mesh=plsc.VectorSubcoreMesh(**_MESH),
        scratch_types=scratch,
    )
    def _prop(src_hbm, dst_hbm, zeros_hbm, *rest):
        tabs_hbm = rest[:F]
        out_hbm = rest[F]
        src_v, dst_v = rest[F + 1], rest[F + 2]
        gbufs = rest[F + 3:F + 3 + F]
        tabs_s = rest[F + 3 + F:F + 3 + 2 * F]
        accs_s = rest[F + 3 + 2 * F:]
        cid = lax.axis_index("c")
        sid = lax.axis_index("s")
        wid = sid * NC + cid
        sl = pl.ds(sid * SEG, SEG)
        for f in range(F):
            pltpu.sync_copy(tabs_hbm[f].at[sl], tabs_s[f].at[sl])
            pltpu.sync_copy(zeros_hbm.at[sl], accs_s[f].at[sl])
        plsc.subcore_barrier()
        base = wid * PER_W

        def body(i, carry):
            off = base + i * CH
            pltpu.sync_copy(src_hbm.at[pl.ds(off, CH)], src_v)
            pltpu.sync_copy(dst_hbm.at[pl.ds(off, CH)], dst_v)
            for f in range(F):
                pltpu.sync_copy(tabs_s[f].at[src_v], gbufs[f])
                pltpu.sync_copy(gbufs[f], accs_s[f].at[dst_v], add=True)
            return carry

        lax.fori_loop(0, NCH, body, 0)
        plsc.subcore_barrier()
        for f in range(F):
            pltpu.sync_copy(accs_s[f].at[sl], out_hbm.at[cid, f, sl])

    return _prop


_prop2 = _make_prop(2)
_prop1 = _make_prop(1)


def _tc1_body(dp_ref, xa0_ref, xb_ref, d_ref, q1a_ref, q1b_ref):
    deg = dp_ref[0] + dp_ref[1] + 1.0
    d = lax.rsqrt(deg)
    d_ref[...] = d
    q1a_ref[...] = d * (2.0 * xa0_ref[...])
    q1b_ref[...] = d * xb_ref[...]


def _tc2_body(d_ref, xa0_ref, xb_ref, up_ref, w1_ref, b1_ref, w2_ref,
              y_ref, qy_ref):
    d = d_ref[...]
    d2 = d * d
    xa = 2.0 * xa0_ref[...]
    xb = xb_ref[...]
    va = d * (up_ref[0, 0] + up_ref[1, 0]) + d2 * xa
    vb = d * (up_ref[0, 1] + up_ref[1, 1]) + d2 * xb
    y = jnp.zeros_like(d)
    for j in range(16):
        h = va * w1_ref[0, j] + vb * w1_ref[1, j] + b1_ref[j]
        h = jnp.maximum(h, 0.0)
        y = y + h * w2_ref[j, 0]
    y_ref[...] = y
    qy_ref[...] = d * y


def _tc3_body(d_ref, y_ref, p2_ref, b2_ref, out_ref):
    d = d_ref[...]
    out_ref[...] = d * (p2_ref[0] + p2_ref[1]) + d * d * y_ref[...] + b2_ref[0]


_PLANE = jax.ShapeDtypeStruct((ROWS, 128), jnp.float32)
_SSPEC = pl.BlockSpec(memory_space=pltpu.SMEM)


def _tc1(dp, xa0, xb):
    return pl.pallas_call(
        _tc1_body, out_shape=(_PLANE, _PLANE, _PLANE),
    )(dp, xa0, xb)


def _tc2(d, xa0, xb, up, W1, b1, W2):
    return pl.pallas_call(
        _tc2_body,
        out_shape=(_PLANE, _PLANE),
        in_specs=[pl.BlockSpec(), pl.BlockSpec(), pl.BlockSpec(),
                  pl.BlockSpec(), _SSPEC, _SSPEC, _SSPEC],
    )(d, xa0, xb, up, W1, b1, W2)


def _tc3(d, y, p2, b2):
    return pl.pallas_call(
        _tc3_body,
        out_shape=_PLANE,
        in_specs=[pl.BlockSpec(), pl.BlockSpec(), pl.BlockSpec(), _SSPEC],
    )(d, y, p2, b2)


def kernel(normalized_x, edge_index, W1, b1, W2, b2):
    src = edge_index[0]
    dst = edge_index[1]
    pad = NPAD - N_NODES
    xa0 = jnp.pad(normalized_x[:, 0], (0, pad)).reshape(ROWS, 128)
    xb = jnp.pad(normalized_x[:, 2], (0, pad)).reshape(ROWS, 128)
    zeros = jnp.zeros((NPAD,), jnp.float32)
    ones = jnp.ones((CH,), jnp.float32)

    dp = _deg_kernel(dst, zeros, ones)
    d, q1a, q1b = _tc1(dp.reshape(NC, ROWS, 128), xa0, xb)
    up = _prop2(src, dst, zeros,
                q1a.reshape(NPAD), q1b.reshape(NPAD))
    y, qy = _tc2(d, xa0, xb, up.reshape(NC, 2, ROWS, 128), W1, b1, W2)
    p2 = _prop1(src, dst, zeros, qy.reshape(NPAD))
    out = _tc3(d, y, p2.reshape(NC, ROWS, 128), b2)
    return out.reshape(NPAD)[:N_NODES].reshape(N_NODES, 1)


# speedup vs baseline: 205.7961x; 1.1275x over previous
import functools

import jax
import jax.numpy as jnp
from jax import lax
from jax.experimental import pallas as pl
from jax.experimental.pallas import tpu as pltpu
from jax.experimental.pallas import tpu_sc as plsc

N_NODES = 100000
N_EDGES = 3200000
NPAD = 100352
ROWS = NPAD // 128
NC, NS = 2, 16
NW = NC * NS
PER_W = N_EDGES // NW
CH = 4000
NCH = PER_W // CH
SEG = NPAD // NS
NI = 4
NG = 2

_MESH = dict(core_axis_name="c", subcore_axis_name="s", num_cores=NC,
             num_subcores=NS)


def _make_prop(F):
    scratch = (
        [pltpu.VMEM((CH,), jnp.int32) for _ in range(2 * NI)]
        + [pltpu.VMEM((CH,), jnp.float32) for _ in range(NG * F)]
        + [pltpu.VMEM_SHARED((NPAD,), jnp.float32) for _ in range(2 * F)]
    )

    @functools.partial(
        pl.kernel,
        out_type=jax.ShapeDtypeStruct((NC, F, NPAD), jnp.float32),
        mesh=plsc.VectorSubcoreMesh(**_MESH),
        scratch_types=scratch,
    )
    def _prop(src_hbm, dst_hbm, zeros_hbm, *rest):
        tabs_hbm = rest[:F]
        out_hbm = rest[F]
        rest = rest[F + 1:]
        src_v = rest[:NI]
        dst_v = rest[NI:2 * NI]
        gbuf = rest[2 * NI:2 * NI + NG * F]
        tabs_s = rest[2 * NI + NG * F:2 * NI + NG * F + F]
        accs_s = rest[2 * NI + NG * F + F:]
        cid = lax.axis_index("c")
        sid = lax.axis_index("s")
        wid = sid * NC + cid
        sl = pl.ds(sid * SEG, SEG)
        for f in range(F):
            pltpu.sync_copy(tabs_hbm[f].at[sl], tabs_s[f].at[sl])
            pltpu.sync_copy(zeros_hbm.at[sl], accs_s[f].at[sl])
        plsc.subcore_barrier()
        base = wid * PER_W

        def _step(k, b):
            bg = b % NG
            off = base + k * CH
            pltpu.sync_copy(src_hbm.at[pl.ds(off, CH)], src_v[b])
            pltpu.sync_copy(dst_hbm.at[pl.ds(off, CH)], dst_v[b])
            for f in range(F):
                pltpu.sync_copy(tabs_s[f].at[src_v[b]], gbuf[bg * F + f])
                pltpu.sync_copy(gbuf[bg * F + f], accs_s[f].at[dst_v[b]],
                                add=True)

        def body(r, carry):
            for b in range(NI):
                _step(r * NI + b, b)
            return carry

        lax.fori_loop(0, NCH // NI, body, 0)
        for k in range(NCH - NCH % NI, NCH):
            _step(k, k % NI)
        plsc.subcore_barrier()
        for f in range(F):
            pltpu.sync_copy(accs_s[f].at[sl], out_hbm.at[cid, f, sl])

    return _prop


_prop2 = _make_prop(2)
_prop1 = _make_prop(1)


def _rb(v):
    return v.astype(jnp.bfloat16).astype(jnp.float32)


def _tc1_body(dp_ref, xa0_ref, xb_ref, d_ref, q1a_ref, q1b_ref):
    deg = dp_ref[0] + dp_ref[1] + 1.0
    d = 1.0 / jnp.sqrt(deg)
    d_ref[...] = d
    q1a_ref[...] = d * _rb(2.0 * xa0_ref[...])
    q1b_ref[...] = d * _rb(xb_ref[...])


def _tc2_body(d_ref, xa0_ref, xb_ref, ua_ref, ub_ref, w1_ref, b1_ref, w2_ref,
              y_ref, qy_ref):
    d = d_ref[...]
    d2 = d * d
    xa = _rb(2.0 * xa0_ref[...])
    xb = _rb(xb_ref[...])
    va = d * (ua_ref[0] + ua_ref[1]) + d2 * xa
    vb = d * (ub_ref[0] + ub_ref[1]) + d2 * xb
    y = jnp.zeros_like(d)
    for j in range(16):
        h = (va * _rb(w1_ref[0, j]) + vb * _rb(w1_ref[1, j]) + b1_ref[j])
        h = jnp.maximum(h, 0.0)
        y = y + _rb(h) * _rb(w2_ref[j, 0])
    y_ref[...] = y
    qy_ref[...] = d * y


def _tc3_body(d_ref, y_ref, p2_ref, b2_ref, out_ref):
    d = d_ref[...]
    out_ref[...] = d * (p2_ref[0] + p2_ref[1]) + d * d * y_ref[...] + b2_ref[0]


_PLANE = jax.ShapeDtypeStruct((ROWS, 128), jnp.float32)
_SSPEC = pl.BlockSpec(memory_space=pltpu.SMEM)


def _tc1(dp, xa0, xb):
    return pl.pallas_call(
        _tc1_body, out_shape=(_PLANE, _PLANE, _PLANE),
    )(dp, xa0, xb)


def _tc2(d, xa0, xb, ua, ub, W1, b1, W2):
    return pl.pallas_call(
        _tc2_body,
        out_shape=(_PLANE, _PLANE),
        in_specs=[pl.BlockSpec()] * 5 + [_SSPEC, _SSPEC, _SSPEC],
    )(d, xa0, xb, ua, ub, W1, b1, W2)


def _tc3(d, y, p2, b2):
    return pl.pallas_call(
        _tc3_body,
        out_shape=_PLANE,
        in_specs=[pl.BlockSpec(), pl.BlockSpec(), pl.BlockSpec(), _SSPEC],
    )(d, y, p2, b2)


def kernel(normalized_x, edge_index, W1, b1, W2, b2):
    src = edge_index[0]
    dst = edge_index[1]
    pad = NPAD - N_NODES
    xa0 = jnp.pad(normalized_x[:, 0], (0, pad)).reshape(ROWS, 128)
    xb = jnp.pad(normalized_x[:, 2], (0, pad)).reshape(ROWS, 128)
    zeros = jnp.zeros((NPAD,), jnp.float32)
    ones = jnp.ones((NPAD,), jnp.float32)

    dp = _prop1(src, dst, zeros, ones)
    d, q1a, q1b = _tc1(dp.reshape(NC, ROWS, 128), xa0, xb)
    up = _prop2(src, dst, zeros,
                q1a.reshape(NPAD), q1b.reshape(NPAD))
    ua = up[:, 0].reshape(NC, ROWS, 128)
    ub = up[:, 1].reshape(NC, ROWS, 128)
    y, qy = _tc2(d, xa0, xb, ua, ub, W1, b1, W2)
    p2 = _prop1(src, dst, zeros, qy.reshape(NPAD))
    out = _tc3(d, y, p2.reshape(NC, ROWS, 128), b2)
    return out.reshape(NPAD)[:N_NODES].reshape(N_NODES, 1)


# speedup vs baseline: 217.9326x; 1.0590x over previous
import functools

import jax
import jax.numpy as jnp
from jax import lax
from jax.experimental import pallas as pl
from jax.experimental.pallas import tpu as pltpu
from jax.experimental.pallas import tpu_sc as plsc

N_NODES = 100000
N_EDGES = 3200000
NPAD = 100352
ROWS = NPAD // 128
NC, NS = 2, 16
NW = NC * NS
PER_W = N_EDGES // NW
CH = 4000
NCH = PER_W // CH
SEG = NPAD // NS
NI = 4
NG = 2

_MESH = dict(core_axis_name="c", subcore_axis_name="s", num_cores=NC,
             num_subcores=NS)


def _make_prop(F):
    scratch = (
        [pltpu.VMEM((CH,), jnp.int32) for _ in range(2 * NI)]
        + [pltpu.VMEM((CH,), jnp.float32) for _ in range(NG * F)]
        + [pltpu.VMEM_SHARED((NPAD,), jnp.float32) for _ in range(2 * F)]
        + [pltpu.SemaphoreType.DMA for _ in range(NI)]
    )

    @functools.partial(
        pl.kernel,
        out_type=jax.ShapeDtypeStruct((NC, F, NPAD), jnp.float32),
        mesh=plsc.VectorSubcoreMesh(**_MESH),
        scratch_types=scratch,
    )
    def _prop(src_hbm, dst_hbm, zeros_hbm, *rest):
        tabs_hbm = rest[:F]
        out_hbm = rest[F]
        rest = rest[F + 1:]
        src_v = rest[:NI]
        dst_v = rest[NI:2 * NI]
        gbuf = rest[2 * NI:2 * NI + NG * F]
        tabs_s = rest[2 * NI + NG * F:2 * NI + NG * F + F]
        accs_s = rest[2 * NI + NG * F + F:2 * NI + NG * F + 2 * F]
        ld_sem = rest[2 * NI + NG * F + 2 * F:]
        cid = lax.axis_index("c")
        sid = lax.axis_index("s")
        wid = sid * NC + cid
        sl = pl.ds(sid * SEG, SEG)
        for f in range(F):
            pltpu.sync_copy(tabs_hbm[f].at[sl], tabs_s[f].at[sl])
            pltpu.sync_copy(zeros_hbm.at[sl], accs_s[f].at[sl])
        plsc.subcore_barrier()
        base = wid * PER_W

        def _start_loads(k, bi):
            off = base + k * CH
            pltpu.async_copy(src_hbm.at[pl.ds(off, CH)], src_v[bi], ld_sem[bi])
            pltpu.async_copy(dst_hbm.at[pl.ds(off, CH)], dst_v[bi], ld_sem[bi])

        def _wait_loads(bi):
            pltpu.make_async_copy(src_hbm.at[pl.ds(0, CH)], src_v[bi],
                                  ld_sem[bi]).wait()
            pltpu.make_async_copy(dst_hbm.at[pl.ds(0, CH)], dst_v[bi],
                                  ld_sem[bi]).wait()

        for b in range(NG):
            _start_loads(b, b)

        def _step(k, b):
            bg = b % NG
            _wait_loads(b)
            for f in range(F):
                pltpu.sync_copy(tabs_s[f].at[src_v[b]], gbuf[bg * F + f])
                pltpu.sync_copy(gbuf[bg * F + f], accs_s[f].at[dst_v[b]],
                                add=True)
            if isinstance(k, int):
                if k + NG < NCH:
                    _start_loads(k + NG, (k + NG) % NI)
            else:
                @pl.when(k + NG < NCH)
                def _():
                    _start_loads(k + NG, (b + NG) % NI)

        def body(r, carry):
            for b in range(NI):
                _step(r * NI + b, b)
            return carry

        lax.fori_loop(0, NCH // NI, body, 0)
        for k in range(NCH - NCH % NI, NCH):
            _step(k, k % NI)
        plsc.subcore_barrier()
        for f in range(F):
            pltpu.sync_copy(accs_s[f].at[sl], out_hbm.at[cid, f, sl])

    return _prop


_prop2 = _make_prop(2)
_prop1 = _make_prop(1)


def _rb(v):
    return v.astype(jnp.bfloat16).astype(jnp.float32)


def _tc1_body(dp_ref, xa0_ref, xb_ref, d_ref, q1a_ref, q1b_ref):
    deg = dp_ref[0] + dp_ref[1] + 1.0
    d = 1.0 / jnp.sqrt(deg)
    d_ref[...] = d
    q1a_ref[...] = d * _rb(2.0 * xa0_ref[...])
    q1b_ref[...] = d * _rb(xb_ref[...])


def _tc2_body(d_ref, xa0_ref, xb_ref, ua_ref, ub_ref, w1_ref, b1_ref, w2_ref,
              y_ref, qy_ref):
    d = d_ref[...]
    d2 = d * d
    xa = _rb(2.0 * xa0_ref[...])
    xb = _rb(xb_ref[...])
    va = d * (ua_ref[0] + ua_ref[1]) + d2 * xa
    vb = d * (ub_ref[0] + ub_ref[1]) + d2 * xb
    y = jnp.zeros_like(d)
    for j in range(16):
        h = (va * _rb(w1_ref[0, j]) + vb * _rb(w1_ref[1, j]) + b1_ref[j])
        h = jnp.maximum(h, 0.0)
        y = y + _rb(h) * _rb(w2_ref[j, 0])
    y_ref[...] = y
    qy_ref[...] = d * y


def _tc3_body(d_ref, y_ref, p2_ref, b2_ref, out_ref):
    d = d_ref[...]
    out_ref[...] = d * (p2_ref[0] + p2_ref[1]) + d * d * y_ref[...] + b2_ref[0]


_PLANE = jax.ShapeDtypeStruct((ROWS, 128), jnp.float32)
_SSPEC = pl.BlockSpec(memory_space=pltpu.SMEM)


def _tc1(dp, xa0, xb):
    return pl.pallas_call(
        _tc1_body, out_shape=(_PLANE, _PLANE, _PLANE),
    )(dp, xa0, xb)


def _tc2(d, xa0, xb, ua, ub, W1, b1, W2):
    return pl.pallas_call(
        _tc2_body,
        out_shape=(_PLANE, _PLANE),
        in_specs=[pl.BlockSpec()] * 5 + [_SSPEC, _SSPEC, _SSPEC],
    )(d, xa0, xb, ua, ub, W1, b1, W2)


def _tc3(d, y, p2, b2):
    return pl.pallas_call(
        _tc3_body,
        out_shape=_PLANE,
        in_specs=[pl.BlockSpec(), pl.BlockSpec(), pl.BlockSpec(), _SSPEC],
    )(d, y, p2, b2)


def kernel(normalized_x, edge_index, W1, b1, W2, b2):
    src = edge_index[0]
    dst = edge_index[1]
    pad = NPAD - N_NODES
    xa0 = jnp.pad(normalized_x[:, 0], (0, pad)).reshape(ROWS, 128)
    xb = jnp.pad(normalized_x[:, 2], (0, pad)).reshape(ROWS, 128)
    zeros = jnp.zeros((NPAD,), jnp.float32)
    ones = jnp.ones((NPAD,), jnp.float32)

    dp = _prop1(src, dst, zeros, ones)
    d, q1a, q1b = _tc1(dp.reshape(NC, ROWS, 128), xa0, xb)
    up = _prop2(src, dst, zeros,
                q1a.reshape(NPAD), q1b.reshape(NPAD))
    ua = up[:, 0].reshape(NC, ROWS, 128)
    ub = up[:, 1].reshape(NC, ROWS, 128)
    y, qy = _tc2(d, xa0, xb, ua, ub, W1, b1, W2)
    p2 = _prop1(src, dst, zeros, qy.reshape(NPAD))
    out = _tc3(d, y, p2.reshape(NC, ROWS, 128), b2)
    return out.reshape(NPAD)[:N_NODES].reshape(N_NODES, 1)
